# baseline (device time: 230643 ns/iter reference)
import jax
import jax.numpy as jnp
from jax import lax
from jax.experimental import pallas as pl
from jax.experimental.pallas import tpu as pltpu

M = 16384
N_OUT = 1024
HALF = M // 2
CR = 256
NCH = HALF // CR


def kernel(x):
    def body(
        x_hbm,
        out_hbm,
        dummy_hbm,
        sbuf,
        rybuf,
        rxbuf,
        ldbuf,
        obuf,
        sem_sy,
        sem_ry,
        sem_sx,
        sem_rx,
        sem_load,
        sem_store,
    ):
        my_x = lax.axis_index("x")
        my_y = lax.axis_index("y")
        ynbr = (my_x, 1 - my_y)
        xnbr = (1 - my_x, my_y)

        barrier = pltpu.get_barrier_semaphore()
        for nbr in (ynbr, xnbr):
            pl.semaphore_signal(
                barrier, inc=1, device_id=nbr,
                device_id_type=pl.DeviceIdType.MESH,
            )
        pl.semaphore_wait(barrier, 2)

        row0 = my_x * HALF
        other0 = HALF - row0
        col_peer = (1 - my_y) * N_OUT
        col_mine = my_y * N_OUT

        def load(slot, rbase, col):
            cp = pltpu.make_async_copy(
                x_hbm.at[0, pl.ds(rbase, CR), pl.ds(col, N_OUT)],
                ldbuf.at[slot],
                sem_load.at[slot],
            )
            cp.start()
            return cp

        st_cps = [None, None, None, None]

        def fold(c, slot, rbase, rbuf):
            ld_cps[slot].wait()
            if st_cps[slot] is not None:
                st_cps[slot].wait()
            obuf[slot] = (ldbuf[slot] + rbuf[c].astype(jnp.float32)).astype(
                jnp.bfloat16
            )
            st = pltpu.make_async_copy(
                obuf.at[slot],
                out_hbm.at[pl.ds(rbase + c * CR, CR), :],
                sem_store.at[slot],
            )
            st.start()
            st_cps[slot] = st

        rd_y = []
        ld_cps = [load(0, row0, col_peer), None, None, None]
        for c in range(NCH):
            s = c % 2
            if c + 1 < NCH:
                ld_cps[1 - s] = load(1 - s, row0 + (c + 1) * CR, col_peer)
            ld_cps[s].wait()
            sbuf[c] = ldbuf[s].astype(jnp.bfloat16)
            r = pltpu.make_async_remote_copy(
                src_ref=sbuf.at[c],
                dst_ref=rybuf.at[c],
                send_sem=sem_sy.at[c],
                recv_sem=sem_ry.at[c],
                device_id=ynbr,
                device_id_type=pl.DeviceIdType.MESH,
            )
            r.start()
            rd_y.append(r)

        rd_x = []
        ld_cps = [load(0, row0, col_mine), None, load(2, other0, col_mine), None]
        for c in range(NCH):
            s = c % 2
            rd_y[c].wait_recv()
            fw = pltpu.make_async_remote_copy(
                src_ref=rybuf.at[c],
                dst_ref=rxbuf.at[c],
                send_sem=sem_sx.at[c],
                recv_sem=sem_rx.at[c],
                device_id=xnbr,
                device_id_type=pl.DeviceIdType.MESH,
            )
            fw.start()
            rd_x.append(fw)
            if c + 1 < NCH:
                ld_cps[1 - s] = load(1 - s, row0 + (c + 1) * CR, col_mine)
            fold(c, s, row0, rybuf)
            if c >= 1:
                cx = c - 1
                sx = 2 + (cx % 2)
                rd_x[cx].wait_recv()
                if cx + 1 < NCH:
                    ld_cps[2 + (1 - cx % 2)] = load(
                        2 + (1 - cx % 2), other0 + (cx + 1) * CR, col_mine
                    )
                fold(cx, sx, other0, rxbuf)
        cx = NCH - 1
        rd_x[cx].wait_recv()
        fold(cx, 2 + (cx % 2), other0, rxbuf)

        for c in range(NCH):
            rd_y[c].wait_send()
            rd_x[c].wait_send()
        for st in st_cps:
            if st is not None:
                st.wait()

    out, _ = pl.pallas_call(
        body,
        out_shape=(
            jax.ShapeDtypeStruct((M, N_OUT), jnp.bfloat16),
            jax.ShapeDtypeStruct((8, 128), jnp.bfloat16),
        ),
        in_specs=[pl.BlockSpec(memory_space=pl.ANY)],
        out_specs=(
            pl.BlockSpec(memory_space=pl.ANY),
            pl.BlockSpec(memory_space=pl.ANY),
        ),
        scratch_shapes=[
            pltpu.VMEM((NCH, CR, N_OUT), jnp.bfloat16),
            pltpu.VMEM((NCH, CR, N_OUT), jnp.bfloat16),
            pltpu.VMEM((NCH, CR, N_OUT), jnp.bfloat16),
            pltpu.VMEM((4, CR, N_OUT), jnp.float32),
            pltpu.VMEM((4, CR, N_OUT), jnp.bfloat16),
            pltpu.SemaphoreType.DMA((NCH,)),
            pltpu.SemaphoreType.DMA((NCH,)),
            pltpu.SemaphoreType.DMA((NCH,)),
            pltpu.SemaphoreType.DMA((NCH,)),
            pltpu.SemaphoreType.DMA((4,)),
            pltpu.SemaphoreType.DMA((4,)),
        ],
        compiler_params=pltpu.CompilerParams(
            collective_id=0, vmem_limit_bytes=100 * 1024 * 1024
        ),
    )(x)
    return out


# device time: 223431 ns/iter; 1.0323x vs baseline; 1.0323x over previous
import jax
import jax.numpy as jnp
from jax import lax
from jax.experimental import pallas as pl
from jax.experimental.pallas import tpu as pltpu

M = 16384
N_OUT = 1024
HALF = M // 2
CRMAX = 512

_SIZES = [256, 256] + [512] * 15
assert sum(_SIZES) == HALF
_OFFS = [sum(_SIZES[:i]) for i in range(len(_SIZES))]
CHUNKS = list(zip(_OFFS, _SIZES))
NCH = len(CHUNKS)


def kernel(x):
    def body(
        x_hbm,
        out_hbm,
        sbuf,
        rybuf,
        rxbuf,
        ldbuf,
        obuf,
        sem_sy,
        sem_ry,
        sem_sx,
        sem_rx,
        sem_load,
        sem_store,
    ):
        my_x = lax.axis_index("x")
        my_y = lax.axis_index("y")
        ynbr = (my_x, 1 - my_y)
        xnbr = (1 - my_x, my_y)

        barrier = pltpu.get_barrier_semaphore()
        for nbr in (ynbr, xnbr):
            pl.semaphore_signal(
                barrier, inc=1, device_id=nbr,
                device_id_type=pl.DeviceIdType.MESH,
            )
        pl.semaphore_wait(barrier, 2)

        row0 = my_x * HALF
        other0 = HALF - row0
        col_peer = (1 - my_y) * N_OUT
        col_mine = my_y * N_OUT

        def load(slot, c, rbase, col):
            off, size = CHUNKS[c]
            cp = pltpu.make_async_copy(
                x_hbm.at[0, pl.ds(rbase + off, size), pl.ds(col, N_OUT)],
                ldbuf.at[slot, pl.ds(0, size)],
                sem_load.at[slot],
            )
            cp.start()
            return cp

        st_cps = [None, None, None, None]

        def fold(c, slot, rbase, rbuf):
            off, size = CHUNKS[c]
            ld_cps[slot].wait()
            if st_cps[slot] is not None:
                st_cps[slot].wait()
            obuf[slot, :size] = (
                ldbuf[slot, :size] + rbuf[off:off + size].astype(jnp.float32)
            ).astype(jnp.bfloat16)
            st = pltpu.make_async_copy(
                obuf.at[slot, pl.ds(0, size)],
                out_hbm.at[pl.ds(rbase + off, size), :],
                sem_store.at[slot],
            )
            st.start()
            st_cps[slot] = st

        rd_y = []
        ld_cps = [load(0, 0, row0, col_peer), None, None, None]
        for c in range(NCH):
            s = c % 2
            if c + 1 < NCH:
                ld_cps[1 - s] = load(1 - s, c + 1, row0, col_peer)
            ld_cps[s].wait()
            off, size = CHUNKS[c]
            sbuf[off:off + size] = ldbuf[s, :size].astype(jnp.bfloat16)
            r = pltpu.make_async_remote_copy(
                src_ref=sbuf.at[pl.ds(off, size)],
                dst_ref=rybuf.at[pl.ds(off, size)],
                send_sem=sem_sy.at[c],
                recv_sem=sem_ry.at[c],
                device_id=ynbr,
                device_id_type=pl.DeviceIdType.MESH,
            )
            r.start()
            rd_y.append(r)

        rd_x = []
        ld_cps = [
            load(0, 0, row0, col_mine), None,
            load(2, 0, other0, col_mine), None,
        ]
        for c in range(NCH):
            s = c % 2
            off, size = CHUNKS[c]
            rd_y[c].wait_recv()
            fw = pltpu.make_async_remote_copy(
                src_ref=rybuf.at[pl.ds(off, size)],
                dst_ref=rxbuf.at[pl.ds(off, size)],
                send_sem=sem_sx.at[c],
                recv_sem=sem_rx.at[c],
                device_id=xnbr,
                device_id_type=pl.DeviceIdType.MESH,
            )
            fw.start()
            rd_x.append(fw)
            if c + 1 < NCH:
                ld_cps[1 - s] = load(1 - s, c + 1, row0, col_mine)
            fold(c, s, row0, rybuf)
            if c >= 1:
                cx = c - 1
                sx = 2 + (cx % 2)
                rd_x[cx].wait_recv()
                if cx + 1 < NCH:
                    ld_cps[2 + (1 - cx % 2)] = load(
                        2 + (1 - cx % 2), cx + 1, other0, col_mine
                    )
                fold(cx, sx, other0, rxbuf)
        cx = NCH - 1
        rd_x[cx].wait_recv()
        fold(cx, 2 + (cx % 2), other0, rxbuf)

        for c in range(NCH):
            rd_y[c].wait_send()
            rd_x[c].wait_send()
        for st in st_cps:
            if st is not None:
                st.wait()

    return pl.pallas_call(
        body,
        out_shape=jax.ShapeDtypeStruct((M, N_OUT), jnp.bfloat16),
        in_specs=[pl.BlockSpec(memory_space=pl.ANY)],
        out_specs=pl.BlockSpec(memory_space=pl.ANY),
        scratch_shapes=[
            pltpu.VMEM((HALF, N_OUT), jnp.bfloat16),
            pltpu.VMEM((HALF, N_OUT), jnp.bfloat16),
            pltpu.VMEM((HALF, N_OUT), jnp.bfloat16),
            pltpu.VMEM((4, CRMAX, N_OUT), jnp.float32),
            pltpu.VMEM((4, CRMAX, N_OUT), jnp.bfloat16),
            pltpu.SemaphoreType.DMA((NCH,)),
            pltpu.SemaphoreType.DMA((NCH,)),
            pltpu.SemaphoreType.DMA((NCH,)),
            pltpu.SemaphoreType.DMA((NCH,)),
            pltpu.SemaphoreType.DMA((4,)),
            pltpu.SemaphoreType.DMA((4,)),
        ],
        compiler_params=pltpu.CompilerParams(
            collective_id=0, vmem_limit_bytes=100 * 1024 * 1024
        ),
    )(x)


# device time: 222709 ns/iter; 1.0356x vs baseline; 1.0032x over previous
import jax
import jax.numpy as jnp
from jax import lax
from jax.experimental import pallas as pl
from jax.experimental.pallas import tpu as pltpu

M = 16384
N_OUT = 1024
HALF = M // 2
CRMAX = 512

_SIZES = [512] * 16
assert sum(_SIZES) == HALF
_OFFS = [sum(_SIZES[:i]) for i in range(len(_SIZES))]
CHUNKS = list(zip(_OFFS, _SIZES))
NCH = len(CHUNKS)


def kernel(x):
    def body(
        x_hbm,
        out_hbm,
        sbuf,
        rybuf,
        rxbuf,
        ldbuf,
        obuf,
        sem_sy,
        sem_ry,
        sem_sx,
        sem_rx,
        sem_load,
        sem_store,
    ):
        my_x = lax.axis_index("x")
        my_y = lax.axis_index("y")
        ynbr = (my_x, 1 - my_y)
        xnbr = (1 - my_x, my_y)

        barrier = pltpu.get_barrier_semaphore()
        for nbr in (ynbr, xnbr):
            pl.semaphore_signal(
                barrier, inc=1, device_id=nbr,
                device_id_type=pl.DeviceIdType.MESH,
            )
        pl.semaphore_wait(barrier, 2)

        row0 = my_x * HALF
        other0 = HALF - row0
        col_peer = (1 - my_y) * N_OUT
        col_mine = my_y * N_OUT

        def load(slot, c, rbase, col):
            off, size = CHUNKS[c]
            cp = pltpu.make_async_copy(
                x_hbm.at[0, pl.ds(rbase + off, size), pl.ds(col, N_OUT)],
                ldbuf.at[slot, pl.ds(0, size)],
                sem_load.at[slot],
            )
            cp.start()
            return cp

        st_cps = [None, None, None, None]

        def fold(c, slot, rbase, rbuf):
            off, size = CHUNKS[c]
            ld_cps[slot].wait()
            if st_cps[slot] is not None:
                st_cps[slot].wait()
            obuf[slot, :size] = (
                ldbuf[slot, :size] + rbuf[off:off + size].astype(jnp.float32)
            ).astype(jnp.bfloat16)
            st = pltpu.make_async_copy(
                obuf.at[slot, pl.ds(0, size)],
                out_hbm.at[pl.ds(rbase + off, size), :],
                sem_store.at[slot],
            )
            st.start()
            st_cps[slot] = st

        rd_y = []
        ld_cps = [load(0, 0, row0, col_peer), None, None, None]
        for c in range(NCH):
            s = c % 2
            if c + 1 < NCH:
                ld_cps[1 - s] = load(1 - s, c + 1, row0, col_peer)
            ld_cps[s].wait()
            off, size = CHUNKS[c]
            sbuf[off:off + size] = ldbuf[s, :size].astype(jnp.bfloat16)
            r = pltpu.make_async_remote_copy(
                src_ref=sbuf.at[pl.ds(off, size)],
                dst_ref=rybuf.at[pl.ds(off, size)],
                send_sem=sem_sy.at[c],
                recv_sem=sem_ry.at[c],
                device_id=ynbr,
                device_id_type=pl.DeviceIdType.MESH,
            )
            r.start()
            rd_y.append(r)

        rd_x = []
        ld_cps = [
            load(0, 0, row0, col_mine), None,
            load(2, 0, other0, col_mine), None,
        ]
        for c in range(NCH):
            s = c % 2
            off, size = CHUNKS[c]
            rd_y[c].wait_recv()
            fw = pltpu.make_async_remote_copy(
                src_ref=rybuf.at[pl.ds(off, size)],
                dst_ref=rxbuf.at[pl.ds(off, size)],
                send_sem=sem_sx.at[c],
                recv_sem=sem_rx.at[c],
                device_id=xnbr,
                device_id_type=pl.DeviceIdType.MESH,
            )
            fw.start()
            rd_x.append(fw)
            if c + 1 < NCH:
                ld_cps[1 - s] = load(1 - s, c + 1, row0, col_mine)
            fold(c, s, row0, rybuf)
            if c >= 1:
                cx = c - 1
                sx = 2 + (cx % 2)
                rd_x[cx].wait_recv()
                if cx + 1 < NCH:
                    ld_cps[2 + (1 - cx % 2)] = load(
                        2 + (1 - cx % 2), cx + 1, other0, col_mine
                    )
                fold(cx, sx, other0, rxbuf)
        cx = NCH - 1
        rd_x[cx].wait_recv()
        fold(cx, 2 + (cx % 2), other0, rxbuf)

        for c in range(NCH):
            rd_y[c].wait_send()
            rd_x[c].wait_send()
        for st in st_cps:
            if st is not None:
                st.wait()

    return pl.pallas_call(
        body,
        out_shape=jax.ShapeDtypeStruct((M, N_OUT), jnp.bfloat16),
        in_specs=[pl.BlockSpec(memory_space=pl.ANY)],
        out_specs=pl.BlockSpec(memory_space=pl.ANY),
        scratch_shapes=[
            pltpu.VMEM((HALF, N_OUT), jnp.bfloat16),
            pltpu.VMEM((HALF, N_OUT), jnp.bfloat16),
            pltpu.VMEM((HALF, N_OUT), jnp.bfloat16),
            pltpu.VMEM((4, CRMAX, N_OUT), jnp.float32),
            pltpu.VMEM((4, CRMAX, N_OUT), jnp.bfloat16),
            pltpu.SemaphoreType.DMA((NCH,)),
            pltpu.SemaphoreType.DMA((NCH,)),
            pltpu.SemaphoreType.DMA((NCH,)),
            pltpu.SemaphoreType.DMA((NCH,)),
            pltpu.SemaphoreType.DMA((4,)),
            pltpu.SemaphoreType.DMA((4,)),
        ],
        compiler_params=pltpu.CompilerParams(
            collective_id=0, vmem_limit_bytes=100 * 1024 * 1024
        ),
    )(x)
